# Initial kernel scaffold; baseline (speedup 1.0000x reference)
import jax
import jax.numpy as jnp
from jax.experimental import pallas as pl


def kernel(sequence_tokens, sequence_embed_weight):
    raise NotImplementedError("write your pallas kernel here")


# speedup vs baseline: 3.1579x; 3.1579x over previous
import functools

import jax
import jax.numpy as jnp
from jax import lax
from jax.experimental import pallas as pl
from jax.experimental.pallas import tpu as pltpu
from jax.experimental.pallas import tpu_sc as plsc

_VOCAB = 30
_D = 2048
_B = 4 * 8192


def _sc_embed(tokens_flat, table):
  info = plsc.get_sparse_core_info()
  nc, ns = info.num_cores, info.num_subcores
  nw = nc * ns
  bpw = _B // nw
  mesh = plsc.VectorSubcoreMesh(core_axis_name="c", subcore_axis_name="s")

  @functools.partial(
      pl.kernel,
      mesh=mesh,
      out_type=jax.ShapeDtypeStruct((_B, _D), jnp.float32),
      scratch_types=[
          pltpu.VMEM((_VOCAB, _D), jnp.float32),
          pltpu.VMEM((bpw,), jnp.int32),
          pltpu.SemaphoreType.DMA,
      ],
  )
  def k(tokens_hbm, table_hbm, out_hbm, table_v, idx_s, sem):
    wid = lax.axis_index("s") * nc + lax.axis_index("c")
    base = wid * bpw
    pltpu.sync_copy(table_hbm, table_v)
    pltpu.sync_copy(tokens_hbm.at[pl.ds(base, bpw)], idx_s)

    def issue(g, carry):
      vec = idx_s[pl.ds(g * 16, 16)]
      for l in range(16):
        tok = vec[l]
        pltpu.make_async_copy(
            table_v.at[pl.ds(tok, 1)],
            out_hbm.at[pl.ds(base + g * 16 + l, 1)],
            sem,
        ).start()
      return carry

    lax.fori_loop(0, bpw // 16, issue, 0)

    pltpu.make_async_copy(
        out_hbm.at[pl.ds(base, bpw)],
        out_hbm.at[pl.ds(base, bpw)],
        sem,
    ).wait()

  return k(tokens_flat, table)


def kernel(sequence_tokens, sequence_embed_weight):
  b, s = sequence_tokens.shape
  out = _sc_embed(sequence_tokens.reshape(b * s), sequence_embed_weight)
  return out.reshape(b, s, _D)
